# Initial kernel scaffold; baseline (speedup 1.0000x reference)
import functools

import jax
import jax.numpy as jnp
from jax import lax
from jax.experimental import pallas as pl
from jax.experimental.pallas import tpu as pltpu
from jax.experimental.pallas import tpu_sc as plsc

KEHALF = 7.199822675975274
SR_CUTOFF = 5.0
LR_CUTOFF = 10.0
LR_CUTOFF2 = LR_CUTOFF * LR_CUTOFF

N_NODES = 100000
N_EDGES = 6400000

LANES = 16
ROW = 128
NC = 2
NS = 16
NW = NC * NS
CHUNK_ROWS = 32


def _edge_energy(d, qi, qj):
    d2 = d * d
    x = d2 + 1.0
    i = plsc.bitcast(x, jnp.int32)
    i = 0x5F3759DF - lax.shift_right_logical(i, 1)
    y = plsc.bitcast(i, jnp.float32)
    y = y * (1.5 - 0.5 * x * y * y)
    y = y * (1.5 - 0.5 * x * y * y)
    y = y * (1.5 - 0.5 * x * y * y)
    inv_shield = y
    shield = x * y
    inv_d = 1.0 / d
    c2 = 1.0 / LR_CUTOFF2
    c1 = 2.0 / LR_CUTOFF
    e_ord = inv_d + d * c2 - c1
    e_shl = inv_shield + shield * c2 - c1
    t = jnp.minimum(jnp.maximum(d * (2.0 / SR_CUTOFF), 0.0), 1.0)
    t3 = t * t * t
    t4 = t3 * t
    t5 = t4 * t
    sw = 1.0 - 6.0 * t5 + 15.0 * t4 - 10.0 * t3
    e = (KEHALF * qi) * qj * (sw * e_shl + (1.0 - sw) * e_ord)
    return jnp.where(d <= LR_CUTOFF, e, 0.0)


def _build_sc_kernel(n_nodes, n_rows):
    base_rows = n_rows // NW
    extra = n_rows - base_rows * NW
    n_full = base_rows // CHUNK_ROWS
    tail_rows = base_rows - n_full * CHUNK_ROWS
    slc = ((n_nodes + NS - 1) // NS + 7) // 8 * 8
    last_slc = n_nodes - (NS - 1) * slc
    assert last_slc > 0 and extra <= NW

    mesh = plsc.VectorSubcoreMesh(core_axis_name="c", subcore_axis_name="s")

    def body(d_hbm, qa_hbm, ii_hbm, jj_hbm, out_hbm,
             qa_v, d_v, ii_v, jj_v, e_v, acc_sh):
        c = lax.axis_index("c")
        s = lax.axis_index("s")
        wid = s * NC + c

        def zero_body(i, carry):
            qa_v[pl.ds(i * LANES, LANES)] = jnp.zeros((LANES,), jnp.float32)
            return carry
        lax.fori_loop(0, slc // LANES + 1, zero_body, 0)

        @pl.when(s < NS - 1)
        def _():
            pltpu.sync_copy(qa_v.at[pl.ds(0, slc)],
                            acc_sh.at[pl.ds(s * slc, slc)])

        @pl.when(s == NS - 1)
        def _():
            pltpu.sync_copy(qa_v.at[pl.ds(0, last_slc)],
                            acc_sh.at[pl.ds((NS - 1) * slc, last_slc)])

        pltpu.sync_copy(qa_hbm, qa_v)
        plsc.subcore_barrier()

        row_base = wid * base_rows + jnp.minimum(wid, extra)

        def process(row0, nr):
            pltpu.sync_copy(d_hbm.at[pl.ds(row0, nr)], d_v.at[pl.ds(0, nr)])
            pltpu.sync_copy(ii_hbm.at[pl.ds(row0, nr)], ii_v.at[pl.ds(0, nr)])
            pltpu.sync_copy(jj_hbm.at[pl.ds(row0, nr)], jj_v.at[pl.ds(0, nr)])

            def row_body(r, carry):
                for v in range(ROW // LANES):
                    sl = pl.ds(v * LANES, LANES)
                    d = d_v[r, sl]
                    qi = plsc.load_gather(qa_v, [ii_v[r, sl]])
                    qj = plsc.load_gather(qa_v, [jj_v[r, sl]])
                    e_v[r, sl] = _edge_energy(d, qi, qj)
                return carry
            lax.fori_loop(0, nr, row_body, 0)

            def scat_body(r, carry):
                pltpu.sync_copy(e_v.at[r], acc_sh.at[ii_v.at[r]], add=True)
                return carry
            lax.fori_loop(0, nr, scat_body, 0)

        if n_full > 0:
            def chunk_body(k, carry):
                process(row_base + k * CHUNK_ROWS, CHUNK_ROWS)
                return carry
            lax.fori_loop(0, n_full, chunk_body, 0)
        if tail_rows > 0:
            process(row_base + n_full * CHUNK_ROWS, tail_rows)
        if extra > 0:
            @pl.when(wid < extra)
            def _():
                process(row_base + base_rows, 1)

        plsc.subcore_barrier()

        @pl.when(s < NS - 1)
        def _():
            pltpu.sync_copy(acc_sh.at[pl.ds(s * slc, slc)],
                            qa_v.at[pl.ds(0, slc)])
            pltpu.sync_copy(qa_v.at[pl.ds(0, slc)],
                            out_hbm.at[c, pl.ds(s * slc, slc)])

        @pl.when(s == NS - 1)
        def _():
            pltpu.sync_copy(acc_sh.at[pl.ds((NS - 1) * slc, last_slc)],
                            qa_v.at[pl.ds(0, last_slc)])
            pltpu.sync_copy(qa_v.at[pl.ds(0, last_slc)],
                            out_hbm.at[c, pl.ds((NS - 1) * slc, last_slc)])

    return pl.kernel(
        body,
        out_type=jax.ShapeDtypeStruct((NC, n_nodes), jnp.float32),
        mesh=mesh,
        scratch_types=[
            pltpu.VMEM((n_nodes,), jnp.float32),
            pltpu.VMEM((CHUNK_ROWS, ROW), jnp.float32),
            pltpu.VMEM((CHUNK_ROWS, ROW), jnp.int32),
            pltpu.VMEM((CHUNK_ROWS, ROW), jnp.int32),
            pltpu.VMEM((CHUNK_ROWS, ROW), jnp.float32),
            pltpu.VMEM_SHARED((n_nodes,), jnp.float32),
        ],
    )


_sc_kernel = _build_sc_kernel(N_NODES, N_EDGES // ROW)


def _combine_body(p_ref, o_ref):
    o_ref[...] = p_ref[0, :] + p_ref[1, :]


def kernel(Dij, Qa, idx_i, idx_j):
    n_rows = N_EDGES // ROW
    d2 = Dij.reshape(n_rows, ROW)
    ii2 = idx_i.reshape(n_rows, ROW)
    jj2 = idx_j.reshape(n_rows, ROW)
    partial = _sc_kernel(d2, Qa, ii2, jj2)
    out = pl.pallas_call(
        _combine_body,
        out_shape=jax.ShapeDtypeStruct((N_NODES,), jnp.float32),
    )(partial)
    return out


# speedup vs baseline: 287.1340x; 287.1340x over previous
import functools

import jax
import jax.numpy as jnp
from jax import lax
from jax.experimental import pallas as pl
from jax.experimental.pallas import tpu as pltpu
from jax.experimental.pallas import tpu_sc as plsc

KEHALF = 7.199822675975274
SR_CUTOFF = 5.0
LR_CUTOFF = 10.0
LR_CUTOFF2 = LR_CUTOFF * LR_CUTOFF

N_NODES = 100000
N_EDGES = 6400000

LANES = 16
ROW = 128
NC = 2
NS = 16
NW = NC * NS
CHUNK_ROWS = 32


def _edge_energy(d, qi, qj):
    d2 = d * d
    x = d2 + 1.0
    i = plsc.bitcast(x, jnp.int32)
    i = 0x5F3759DF - lax.shift_right_logical(i, 1)
    y = plsc.bitcast(i, jnp.float32)
    y = y * (1.5 - 0.5 * x * y * y)
    y = y * (1.5 - 0.5 * x * y * y)
    y = y * (1.5 - 0.5 * x * y * y)
    inv_shield = y
    shield = x * y
    inv_d = 1.0 / d
    c2 = 1.0 / LR_CUTOFF2
    c1 = 2.0 / LR_CUTOFF
    e_ord = inv_d + d * c2 - c1
    e_shl = inv_shield + shield * c2 - c1
    t = jnp.minimum(jnp.maximum(d * (2.0 / SR_CUTOFF), 0.0), 1.0)
    t3 = t * t * t
    t4 = t3 * t
    t5 = t4 * t
    sw = 1.0 - 6.0 * t5 + 15.0 * t4 - 10.0 * t3
    e = (KEHALF * qi) * qj * (sw * e_shl + (1.0 - sw) * e_ord)
    return jnp.where(d <= LR_CUTOFF, e, 0.0)


def _build_sc_kernel(n_nodes, n_rows):
    assert n_rows % 8 == 0
    blocks8 = n_rows // 8
    base_rows = (blocks8 // NW) * 8
    extra = blocks8 - (base_rows // 8) * NW
    n_full = base_rows // CHUNK_ROWS
    tail_rows = base_rows - n_full * CHUNK_ROWS
    assert tail_rows + 8 <= CHUNK_ROWS or extra == 0
    slc = ((n_nodes + NS - 1) // NS + 7) // 8 * 8
    last_slc = n_nodes - (NS - 1) * slc
    assert last_slc > 0 and extra <= NW

    mesh = plsc.VectorSubcoreMesh(core_axis_name="c", subcore_axis_name="s")

    def body(d_hbm, qa_hbm, ii_hbm, jj_hbm, out_hbm,
             qa_v, d_v, ii_v, jj_v, e_v, acc_sh):
        c = lax.axis_index("c")
        s = lax.axis_index("s")
        wid = s * NC + c

        def zero_body(i, carry):
            qa_v[pl.ds(i * LANES, LANES)] = jnp.zeros((LANES,), jnp.float32)
            return carry
        lax.fori_loop(0, slc // LANES + 1, zero_body, 0)

        @pl.when(s < NS - 1)
        def _():
            pltpu.sync_copy(qa_v.at[pl.ds(0, slc)],
                            acc_sh.at[pl.ds(s * slc, slc)])

        @pl.when(s == NS - 1)
        def _():
            pltpu.sync_copy(qa_v.at[pl.ds(0, last_slc)],
                            acc_sh.at[pl.ds((NS - 1) * slc, last_slc)])

        pltpu.sync_copy(qa_hbm, qa_v)
        plsc.subcore_barrier()

        row_base = wid * base_rows + 8 * jnp.minimum(wid, extra)

        def process(row0, nr):
            pltpu.sync_copy(d_hbm.at[pl.ds(row0, nr)], d_v.at[pl.ds(0, nr)])
            pltpu.sync_copy(ii_hbm.at[pl.ds(row0, nr)], ii_v.at[pl.ds(0, nr)])
            pltpu.sync_copy(jj_hbm.at[pl.ds(row0, nr)], jj_v.at[pl.ds(0, nr)])

            def row_body(r, carry):
                for v in range(ROW // LANES):
                    sl = pl.ds(v * LANES, LANES)
                    d = d_v[r, sl]
                    qi = plsc.load_gather(qa_v, [ii_v[r, sl]])
                    qj = plsc.load_gather(qa_v, [jj_v[r, sl]])
                    e_v[r, sl] = _edge_energy(d, qi, qj)
                return carry
            lax.fori_loop(0, nr, row_body, 0)

            def scat_body(r, carry):
                pltpu.sync_copy(e_v.at[r], acc_sh.at[ii_v.at[r]], add=True)
                return carry
            lax.fori_loop(0, nr, scat_body, 0)

        if n_full > 0:
            def chunk_body(k, carry):
                process(row_base + k * CHUNK_ROWS, CHUNK_ROWS)
                return carry
            lax.fori_loop(0, n_full, chunk_body, 0)
        if extra > 0:
            @pl.when(wid < extra)
            def _():
                process(row_base + n_full * CHUNK_ROWS, tail_rows + 8)
            if tail_rows > 0:
                @pl.when(wid >= extra)
                def _():
                    process(row_base + n_full * CHUNK_ROWS, tail_rows)
        elif tail_rows > 0:
            process(row_base + n_full * CHUNK_ROWS, tail_rows)

        plsc.subcore_barrier()

        @pl.when(s < NS - 1)
        def _():
            pltpu.sync_copy(acc_sh.at[pl.ds(s * slc, slc)],
                            qa_v.at[pl.ds(0, slc)])
            pltpu.sync_copy(qa_v.at[pl.ds(0, slc)],
                            out_hbm.at[pl.ds(c * n_nodes + s * slc, slc)])

        @pl.when(s == NS - 1)
        def _():
            pltpu.sync_copy(acc_sh.at[pl.ds((NS - 1) * slc, last_slc)],
                            qa_v.at[pl.ds(0, last_slc)])
            pltpu.sync_copy(qa_v.at[pl.ds(0, last_slc)],
                            out_hbm.at[pl.ds(c * n_nodes + (NS - 1) * slc, last_slc)])

    return pl.kernel(
        body,
        out_type=jax.ShapeDtypeStruct((NC * n_nodes,), jnp.float32),
        mesh=mesh,
        compiler_params=pltpu.CompilerParams(needs_layout_passes=False),
        scratch_types=[
            pltpu.VMEM((n_nodes,), jnp.float32),
            pltpu.VMEM((CHUNK_ROWS, ROW), jnp.float32),
            pltpu.VMEM((CHUNK_ROWS, ROW), jnp.int32),
            pltpu.VMEM((CHUNK_ROWS, ROW), jnp.int32),
            pltpu.VMEM((CHUNK_ROWS, ROW), jnp.float32),
            pltpu.VMEM_SHARED((n_nodes,), jnp.float32),
        ],
    )


_sc_kernel = _build_sc_kernel(N_NODES, N_EDGES // ROW)


def _combine_body(p_ref, o_ref):
    o_ref[...] = p_ref[0, :] + p_ref[1, :]


def kernel(Dij, Qa, idx_i, idx_j):
    n_rows = N_EDGES // ROW
    d2 = Dij.reshape(n_rows, ROW)
    ii2 = idx_i.reshape(n_rows, ROW)
    jj2 = idx_j.reshape(n_rows, ROW)
    partial = _sc_kernel(d2, Qa, ii2, jj2).reshape(NC, N_NODES)
    out = pl.pallas_call(
        _combine_body,
        out_shape=jax.ShapeDtypeStruct((N_NODES,), jnp.float32),
    )(partial)
    return out


# speedup vs baseline: 488.6859x; 1.7019x over previous
import functools

import jax
import jax.numpy as jnp
from jax import lax
from jax.experimental import pallas as pl
from jax.experimental.pallas import tpu as pltpu
from jax.experimental.pallas import tpu_sc as plsc

KEHALF = 7.199822675975274
SR_CUTOFF = 5.0
LR_CUTOFF = 10.0
LR_CUTOFF2 = LR_CUTOFF * LR_CUTOFF

N_NODES = 100000
N_EDGES = 6400000

LANES = 16
ROW = 128
NC = 2
NS = 16
NW = NC * NS
CHUNK_ROWS = 16


def _edge_energy(d, qi, qj):
    d2 = d * d
    x = d2 + 1.0
    i = plsc.bitcast(x, jnp.int32)
    i = 0x5F3759DF - lax.shift_right_logical(i, 1)
    y = plsc.bitcast(i, jnp.float32)
    y = y * (1.5 - 0.5 * x * y * y)
    y = y * (1.5 - 0.5 * x * y * y)
    y = y * (1.5 - 0.5 * x * y * y)
    inv_shield = y
    shield = x * y
    inv_d = 1.0 / d
    c2 = 1.0 / LR_CUTOFF2
    c1 = 2.0 / LR_CUTOFF
    e_ord = inv_d + d * c2 - c1
    e_shl = inv_shield + shield * c2 - c1
    t = jnp.minimum(jnp.maximum(d * (2.0 / SR_CUTOFF), 0.0), 1.0)
    t3 = t * t * t
    t4 = t3 * t
    t5 = t4 * t
    sw = 1.0 - 6.0 * t5 + 15.0 * t4 - 10.0 * t3
    e = (KEHALF * qi) * qj * (sw * e_shl + (1.0 - sw) * e_ord)
    return jnp.where(d <= LR_CUTOFF, e, 0.0)


def _build_sc_kernel(n_nodes, n_rows):
    assert n_rows % 8 == 0
    blocks8 = n_rows // 8
    base_rows = (blocks8 // NW) * 8
    extra = blocks8 - (base_rows // 8) * NW
    n_full = base_rows // CHUNK_ROWS
    n_pipe = n_full - (n_full % 2)
    rem = base_rows - n_pipe * CHUNK_ROWS

    def _split_tail(r):
        out = []
        while r > 0:
            t = min(CHUNK_ROWS, r)
            out.append(t)
            r -= t
        return out

    tail_plain = _split_tail(rem)
    tail_bonus = _split_tail(rem + 8)
    slc = ((n_nodes + NS - 1) // NS + 7) // 8 * 8
    last_slc = n_nodes - (NS - 1) * slc
    assert last_slc > 0 and extra <= NW

    mesh = plsc.VectorSubcoreMesh(core_axis_name="c", subcore_axis_name="s")

    def body(d_hbm, qa_hbm, ii_hbm, jj_hbm, out_hbm,
             qa_v, d_v, ii_v, jj_v, e_v, acc_sh, sem_in, sem_scat):
        c = lax.axis_index("c")
        s = lax.axis_index("s")
        wid = s * NC + c
        row_base = wid * base_rows + 8 * jnp.minimum(wid, extra)

        def stage(row0, b):
            pltpu.async_copy(d_hbm.at[pl.ds(row0, CHUNK_ROWS)], d_v.at[b], sem_in)
            pltpu.async_copy(ii_hbm.at[pl.ds(row0, CHUNK_ROWS)], ii_v.at[b], sem_in)
            pltpu.async_copy(jj_hbm.at[pl.ds(row0, CHUNK_ROWS)], jj_v.at[b], sem_in)

        def wait_stage(b):
            pltpu.make_async_copy(d_hbm.at[pl.ds(0, CHUNK_ROWS)], d_v.at[b], sem_in).wait()
            pltpu.make_async_copy(ii_hbm.at[pl.ds(0, CHUNK_ROWS)], ii_v.at[b], sem_in).wait()
            pltpu.make_async_copy(jj_hbm.at[pl.ds(0, CHUNK_ROWS)], jj_v.at[b], sem_in).wait()

        def drain_scatter():
            def wait_one(r, carry):
                pltpu.make_async_copy(e_v.at[0, 0], acc_sh.at[ii_v.at[0, 0]],
                                      sem_scat).wait()
                return carry
            lax.fori_loop(0, CHUNK_ROWS, wait_one, 0)

        if n_pipe > 0:
            stage(row_base, 0)

        def zero_body(i, carry):
            qa_v[pl.ds(i * LANES, LANES)] = jnp.zeros((LANES,), jnp.float32)
            return carry
        lax.fori_loop(0, slc // LANES + 1, zero_body, 0)

        @pl.when(s < NS - 1)
        def _():
            pltpu.sync_copy(qa_v.at[pl.ds(0, slc)],
                            acc_sh.at[pl.ds(s * slc, slc)])

        @pl.when(s == NS - 1)
        def _():
            pltpu.sync_copy(qa_v.at[pl.ds(0, last_slc)],
                            acc_sh.at[pl.ds((NS - 1) * slc, last_slc)])

        pltpu.sync_copy(qa_hbm, qa_v)
        plsc.subcore_barrier()

        def compute_rows(b, nr, fire_async):
            def row_body(r, carry):
                for v in range(ROW // LANES):
                    sl = pl.ds(v * LANES, LANES)
                    d = d_v[b, r, sl]
                    qi = plsc.load_gather(qa_v, [ii_v[b, r, sl]])
                    qj = plsc.load_gather(qa_v, [jj_v[b, r, sl]])
                    e_v[b, r, sl] = _edge_energy(d, qi, qj)
                if fire_async:
                    pltpu.async_copy(e_v.at[b, r], acc_sh.at[ii_v.at[b, r]],
                                     sem_scat, add=True)
                return carry
            lax.fori_loop(0, nr, row_body, 0)

        def half_step(k, b, m):
            wait_stage(b)
            if b == 0:
                @pl.when(m >= 1)
                def _():
                    drain_scatter()
            else:
                drain_scatter()
            @pl.when(k < n_pipe - 1)
            def _():
                stage(row_base + (k + 1) * CHUNK_ROWS, 1 - b)
            compute_rows(b, CHUNK_ROWS, fire_async=True)

        if n_pipe > 0:
            def pipe_body(m, carry):
                half_step(2 * m, 0, m)
                half_step(2 * m + 1, 1, m)
                return carry
            lax.fori_loop(0, n_pipe // 2, pipe_body, 0)
            drain_scatter()

        def process(row0, nr):
            pltpu.sync_copy(d_hbm.at[pl.ds(row0, nr)], d_v.at[0, pl.ds(0, nr)])
            pltpu.sync_copy(ii_hbm.at[pl.ds(row0, nr)], ii_v.at[0, pl.ds(0, nr)])
            pltpu.sync_copy(jj_hbm.at[pl.ds(row0, nr)], jj_v.at[0, pl.ds(0, nr)])
            compute_rows(0, nr, fire_async=False)

            def scat_body(r, carry):
                pltpu.sync_copy(e_v.at[0, r], acc_sh.at[ii_v.at[0, r]], add=True)
                return carry
            lax.fori_loop(0, nr, scat_body, 0)

        def run_tail(sizes):
            off = row_base + n_pipe * CHUNK_ROWS
            for t in sizes:
                process(off, t)
                off = off + t

        if extra > 0:
            @pl.when(wid < extra)
            def _():
                run_tail(tail_bonus)
            if tail_plain:
                @pl.when(wid >= extra)
                def _():
                    run_tail(tail_plain)
        elif tail_plain:
            run_tail(tail_plain)

        plsc.subcore_barrier()

        @pl.when(s < NS - 1)
        def _():
            pltpu.sync_copy(acc_sh.at[pl.ds(s * slc, slc)],
                            qa_v.at[pl.ds(0, slc)])
            pltpu.sync_copy(qa_v.at[pl.ds(0, slc)],
                            out_hbm.at[pl.ds(c * n_nodes + s * slc, slc)])

        @pl.when(s == NS - 1)
        def _():
            pltpu.sync_copy(acc_sh.at[pl.ds((NS - 1) * slc, last_slc)],
                            qa_v.at[pl.ds(0, last_slc)])
            pltpu.sync_copy(qa_v.at[pl.ds(0, last_slc)],
                            out_hbm.at[pl.ds(c * n_nodes + (NS - 1) * slc, last_slc)])

    return pl.kernel(
        body,
        out_type=jax.ShapeDtypeStruct((NC * n_nodes,), jnp.float32),
        mesh=mesh,
        compiler_params=pltpu.CompilerParams(needs_layout_passes=False),
        scratch_types=[
            pltpu.VMEM((n_nodes,), jnp.float32),
            pltpu.VMEM((2, CHUNK_ROWS, ROW), jnp.float32),
            pltpu.VMEM((2, CHUNK_ROWS, ROW), jnp.int32),
            pltpu.VMEM((2, CHUNK_ROWS, ROW), jnp.int32),
            pltpu.VMEM((2, CHUNK_ROWS, ROW), jnp.float32),
            pltpu.VMEM_SHARED((n_nodes,), jnp.float32),
            pltpu.SemaphoreType.DMA,
            pltpu.SemaphoreType.DMA,
        ],
    )


_sc_kernel = _build_sc_kernel(N_NODES, N_EDGES // ROW)


def _combine_body(p_ref, o_ref):
    o_ref[...] = p_ref[0, :] + p_ref[1, :]


def kernel(Dij, Qa, idx_i, idx_j):
    n_rows = N_EDGES // ROW
    d2 = Dij.reshape(n_rows, ROW)
    ii2 = idx_i.reshape(n_rows, ROW)
    jj2 = idx_j.reshape(n_rows, ROW)
    partial = _sc_kernel(d2, Qa, ii2, jj2).reshape(NC, N_NODES)
    out = pl.pallas_call(
        _combine_body,
        out_shape=jax.ShapeDtypeStruct((N_NODES,), jnp.float32),
    )(partial)
    return out
